# baseline (speedup 1.0000x reference)
import functools

import jax
import jax.numpy as jnp
from jax import lax
from jax.experimental import pallas as pl
from jax.experimental.pallas import tpu as pltpu
from jax.experimental.pallas import tpu_sc as plsc

NC, NS, LANES = 2, 16, 16
NW = NC * NS
OW = 128


def _pad_tables(tables):
    NF, V, H = tables.shape
    return jnp.pad(tables, ((0, 0), (0, 0), (0, OW - H)))


def _emb_sum(tab128, idx4, P, V, H, NF):
    CHUNK = P // NW
    CPAD = 2048
    SB = 32
    NB = CHUNK // SB
    IR = CPAD // 128

    mesh = plsc.VectorSubcoreMesh(core_axis_name="c", subcore_axis_name="s")

    @functools.partial(
        pl.kernel,
        out_type=jax.ShapeDtypeStruct((P, OW), jnp.float32),
        mesh=mesh,
        scratch_types=(
            [pltpu.VMEM((IR, 128), jnp.int32) for _ in range(NF)]
            + [pltpu.VMEM((SB, OW), jnp.float32) for _ in range(2 * NF)]
            + [pltpu.SemaphoreType.DMA for _ in range(3)]
        ),
    )
    def emb_kernel(tab_hbm, idx_hbm, out_hbm, *scratch):
        idx_v = scratch[:NF]
        rows_v = [scratch[NF:2 * NF], scratch[2 * NF:3 * NF]]
        isem, semA, semB = scratch[3 * NF:]
        sems = [semA, semB]
        wid = lax.axis_index("s") * NC + lax.axis_index("c")
        base0 = wid * CHUNK

        idescs = [
            pltpu.async_copy(
                idx_hbm.at[pl.ds((f * NW + wid) * IR, IR)], idx_v[f], isem
            )
            for f in range(NF)
        ]
        for d in idescs:
            d.wait()

        def fire(b, st):
            r = b // 4
            c = (b % 4) * SB
            return [
                pltpu.async_copy(
                    tab_hbm.at[f].at[idx_v[f].at[r, pl.ds(c, SB)]],
                    rows_v[st][f],
                    sems[st],
                )
                for f in range(NF)
            ]

        def consume(b, st):
            rv = rows_v[st]

            @pl.loop(0, SB)
            def _(p, rv=rv):
                for j in range(H // LANES):
                    sl = pl.ds(j * LANES, LANES)
                    v = rv[0][p, sl]
                    for f in range(1, NF):
                        v = v + rv[f][p, sl]
                    rv[0][p, sl] = v

            pltpu.sync_copy(rv[0], out_hbm.at[pl.ds(base0 + b * SB, SB)])

        def drain(b, st):
            r = b // 4
            c = (b % 4) * SB
            for f in range(NF):
                pltpu.make_async_copy(
                    tab_hbm.at[f].at[idx_v[f].at[r, pl.ds(c, SB)]],
                    rows_v[st][f],
                    sems[st],
                ).wait()

        fire(0, 0)

        @pl.loop(0, NB // 2)
        def _(k):
            b0 = 2 * k
            fire(b0 + 1, 1)
            drain(b0, 0)
            consume(b0, 0)

            @pl.when(k < NB // 2 - 1)
            def _():
                fire(b0 + 2, 0)

            drain(b0 + 1, 1)
            consume(b0 + 1, 1)

    return emb_kernel(tab128, idx4)


def _mlp(emb, emb2, W1, b1, gamma, beta, H, Ll, Bb):
    def body(x_ref, x2_ref, w_ref, b_ref, g_ref, bt_ref, o_ref):
        x = x_ref[:, 0:H] + x2_ref[:, 0:H]
        h = jnp.dot(x, w_ref[...], preferred_element_type=jnp.float32) + b_ref[...]
        mu = jnp.mean(h, axis=-1, keepdims=True)
        var = jnp.mean(jnp.square(h - mu), axis=-1, keepdims=True)
        hn = (h - mu) * lax.rsqrt(var + 1e-5) * g_ref[...] + bt_ref[...]
        o_ref[0] = jnp.maximum(hn, 0.0).T

    return pl.pallas_call(
        body,
        grid=(Ll,),
        in_specs=[
            pl.BlockSpec((Bb, OW), lambda i: (i, 0)),
            pl.BlockSpec((Bb, OW), lambda i: (i, 0)),
            pl.BlockSpec((H, H), lambda i: (0, 0)),
            pl.BlockSpec((1, H), lambda i: (0, 0)),
            pl.BlockSpec((1, H), lambda i: (0, 0)),
            pl.BlockSpec((1, H), lambda i: (0, 0)),
        ],
        out_specs=pl.BlockSpec((1, H, Bb), lambda i: (i, 0, 0)),
        out_shape=jax.ShapeDtypeStruct((Ll, H, Bb), jnp.float32),
    )(emb, emb2, W1, b1.reshape(1, H), gamma.reshape(1, H), beta.reshape(1, H))


def kernel(tables, W1, b1, gamma, beta, indices):
    NF, V, H = tables.shape
    _, Bb, Ll = indices.shape
    P = Bb * Ll
    CHUNK = P // NW
    CPAD = 2048

    idx = jnp.transpose(indices, (0, 2, 1)).reshape(NF, NW, CHUNK).astype(jnp.int32)
    idxp = jnp.pad(idx, ((0, 0), (0, 0), (0, CPAD - CHUNK)))
    NH = NF // 2

    embs = []
    for lo, hi in ((0, NH), (NH, NF)):
        tabh = _pad_tables(tables[lo:hi])
        idx4 = idxp[lo:hi].reshape(-1, 128)
        embs.append(_emb_sum(tabh, idx4, P, V, H, NH))

    out_t = _mlp(embs[0], embs[1], W1, b1, gamma, beta, H, Ll, Bb)
    return jnp.transpose(out_t, (2, 0, 1))


# speedup vs baseline: 1.4474x; 1.4474x over previous
import functools

import jax
import jax.numpy as jnp
from jax import lax
from jax.experimental import pallas as pl
from jax.experimental.pallas import tpu as pltpu
from jax.experimental.pallas import tpu_sc as plsc

NC, NS, LANES = 2, 16, 16
NW = NC * NS
OW = 128


def _pad_tables(tables):
    NF, V, H = tables.shape
    return jnp.pad(tables, ((0, 0), (0, 0), (0, OW - H)))


def _emb_sum(tab128, idx4, P, V, H, NF):
    CHUNK = P // NW
    CPAD = 2048
    SB = 32
    NB = CHUNK // SB
    IR = CPAD // 128

    mesh = plsc.VectorSubcoreMesh(core_axis_name="c", subcore_axis_name="s")

    @functools.partial(
        pl.kernel,
        out_type=jax.ShapeDtypeStruct((P, OW), jnp.float32),
        mesh=mesh,
        scratch_types=(
            [pltpu.VMEM((IR, 128), jnp.int32) for _ in range(NF)]
            + [pltpu.VMEM((SB, OW), jnp.float32) for _ in range(2 * NF)]
            + [pltpu.SemaphoreType.DMA for _ in range(3)]
        ),
    )
    def emb_kernel(tab_hbm, idx_hbm, out_hbm, *scratch):
        idx_v = scratch[:NF]
        rows_v = [scratch[NF:2 * NF], scratch[2 * NF:3 * NF]]
        isem, semA, semB = scratch[3 * NF:]
        sems = [semA, semB]
        wid = lax.axis_index("s") * NC + lax.axis_index("c")
        base0 = wid * CHUNK

        idescs = [
            pltpu.async_copy(
                idx_hbm.at[pl.ds((f * NW + wid) * IR, IR)], idx_v[f], isem
            )
            for f in range(NF)
        ]
        for d in idescs:
            d.wait()

        def fire(b, st):
            r = b // 4
            c = (b % 4) * SB
            return [
                pltpu.async_copy(
                    tab_hbm.at[f].at[idx_v[f].at[r, pl.ds(c, SB)]],
                    rows_v[st][f],
                    sems[st],
                )
                for f in range(NF)
            ]

        def consume(b, st):
            rv = rows_v[st]

            @pl.loop(0, SB)
            def _(p, rv=rv):
                for j in range(H // LANES):
                    sl = pl.ds(j * LANES, LANES)
                    v = rv[0][p, sl]
                    for f in range(1, NF):
                        v = v + rv[f][p, sl]
                    rv[0][p, sl] = v

            pltpu.sync_copy(rv[0], out_hbm.at[pl.ds(base0 + b * SB, SB)])

        def drain(b, st):
            r = b // 4
            c = (b % 4) * SB
            for f in range(NF):
                pltpu.make_async_copy(
                    tab_hbm.at[f].at[idx_v[f].at[r, pl.ds(c, SB)]],
                    rows_v[st][f],
                    sems[st],
                ).wait()

        fire(0, 0)

        @pl.loop(0, NB // 2)
        def _(k):
            b0 = 2 * k
            fire(b0 + 1, 1)
            drain(b0, 0)
            consume(b0, 0)

            @pl.when(k < NB // 2 - 1)
            def _():
                fire(b0 + 2, 0)

            drain(b0 + 1, 1)
            consume(b0 + 1, 1)

    return emb_kernel(tab128, idx4)


def _mlp(emb, W1, b1, gamma, beta, H, Ll, Bb):
    def body(x_ref, w_ref, b_ref, g_ref, bt_ref, o_ref):
        x = x_ref[:, 0:H]
        h = jnp.dot(x, w_ref[...], preferred_element_type=jnp.float32) + b_ref[...]
        mu = jnp.mean(h, axis=-1, keepdims=True)
        var = jnp.mean(jnp.square(h - mu), axis=-1, keepdims=True)
        hn = (h - mu) * lax.rsqrt(var + 1e-5) * g_ref[...] + bt_ref[...]
        o_ref[0] = jnp.maximum(hn, 0.0).T

    return pl.pallas_call(
        body,
        grid=(Ll,),
        in_specs=[
            pl.BlockSpec((Bb, OW), lambda i: (i, 0)),
            pl.BlockSpec((H, H), lambda i: (0, 0)),
            pl.BlockSpec((1, H), lambda i: (0, 0)),
            pl.BlockSpec((1, H), lambda i: (0, 0)),
            pl.BlockSpec((1, H), lambda i: (0, 0)),
        ],
        out_specs=pl.BlockSpec((1, H, Bb), lambda i: (i, 0, 0)),
        out_shape=jax.ShapeDtypeStruct((Ll, H, Bb), jnp.float32),
    )(emb, W1, b1.reshape(1, H), gamma.reshape(1, H), beta.reshape(1, H))


def kernel(tables, W1, b1, gamma, beta, indices):
    NF, V, H = tables.shape
    _, Bb, Ll = indices.shape
    P = Bb * Ll
    CHUNK = P // NW
    CPAD = 2048

    tab128 = _pad_tables(tables)
    idx = jnp.transpose(indices, (0, 2, 1)).reshape(NF, NW, CHUNK).astype(jnp.int32)
    idx4 = jnp.pad(idx, ((0, 0), (0, 0), (0, CPAD - CHUNK))).reshape(-1, 128)

    emb = _emb_sum(tab128, idx4, P, V, H, NF)
    out_t = _mlp(emb, W1, b1, gamma, beta, H, Ll, Bb)
    return jnp.transpose(out_t, (2, 0, 1))
